# baseline (speedup 1.0000x reference)
import functools

import jax
import jax.numpy as jnp
from jax import lax
from jax.experimental import pallas as pl
from jax.experimental.pallas import tpu as pltpu
from jax.experimental.pallas import tpu_sc as plsc

N = 10000
E = 320000
D = 128
G = 64
NC = 2
NS = 16
NW = NC * NS
CHUNK = 128
NCHUNK = 80
HALF = NCHUNK // 2
EPW = CHUNK * NCHUNK
E_PAD = EPW * NW
ACC_ROWS = N + 8
STRIPE = 640
TAIL = N - 15 * STRIPE

@functools.cache
def _make_sc_agg():
    mesh = plsc.VectorSubcoreMesh(core_axis_name="c", subcore_axis_name="s")

    @functools.partial(
        pl.kernel,
        mesh=mesh,
        out_type=jax.ShapeDtypeStruct((NC, N, D), jnp.float32),
        scratch_types=[
            pltpu.VMEM((NCHUNK, CHUNK), jnp.int32),
            pltpu.VMEM((NCHUNK, CHUNK), jnp.int32),
            pltpu.VMEM((CHUNK, D), jnp.float32),
            pltpu.VMEM_SHARED((ACC_ROWS, D), jnp.float32),
            pltpu.SemaphoreType.DMA,
        ],
    )
    def _sc_agg(h_hbm, src_hbm, dst_hbm, zeros_hbm, out_hbm,
                src_v, dst_v, rows_v, acc_sh, semg):
        c = lax.axis_index("c")
        s = lax.axis_index("s")
        w = c * NS + s

        pltpu.async_copy(src_hbm.at[w], src_v, semg)
        pltpu.async_copy(dst_hbm.at[w], dst_v, semg)

        @pl.when(s < 15)
        def _():
            pltpu.sync_copy(zeros_hbm.at[pl.ds(s * STRIPE, STRIPE)],
                            acc_sh.at[pl.ds(s * STRIPE, STRIPE)])

        @pl.when(s == 15)
        def _():
            pltpu.sync_copy(zeros_hbm.at[pl.ds(15 * STRIPE, TAIL)],
                            acc_sh.at[pl.ds(15 * STRIPE, TAIL)])

        pltpu.make_async_copy(src_hbm.at[w], src_v, semg).wait()
        pltpu.make_async_copy(dst_hbm.at[w], dst_v, semg).wait()
        plsc.subcore_barrier()

        def body(i, carry):
            pltpu.async_copy(h_hbm.at[src_v.at[i]], rows_v, semg).wait()
            pltpu.sync_copy(rows_v, acc_sh.at[dst_v.at[i]], add=True)
            return carry

        lax.fori_loop(0, NCHUNK, body, 0)
        plsc.subcore_barrier()

        @pl.when(s < 15)
        def _():
            pltpu.sync_copy(acc_sh.at[pl.ds(s * STRIPE, STRIPE)],
                            out_hbm.at[c, pl.ds(s * STRIPE, STRIPE)])

        @pl.when(s == 15)
        def _():
            pltpu.sync_copy(acc_sh.at[pl.ds(15 * STRIPE, TAIL)],
                            out_hbm.at[c, pl.ds(15 * STRIPE, TAIL)])

    return _sc_agg


def _tc_layer_body(h_ref, agg_ref, w1_ref, b1_ref, w2_ref, b2_ref,
                   gamma_ref, beta_ref, eps_ref, out_ref):
    z = (1.0 + eps_ref[0, 0]) * h_ref[...] + agg_ref[0] + agg_ref[1]
    z = jnp.maximum(
        jnp.dot(z, w1_ref[...], preferred_element_type=jnp.float32)
        + b1_ref[...], 0.0)
    z = jnp.maximum(
        jnp.dot(z, w2_ref[...], preferred_element_type=jnp.float32)
        + b2_ref[...], 0.0)
    mean = jnp.mean(z, axis=0, keepdims=True)
    var = jnp.mean((z - mean) * (z - mean), axis=0, keepdims=True)
    out_ref[...] = ((z - mean) * lax.rsqrt(var + 1e-5) * gamma_ref[...]
                    + beta_ref[...])


_tc_layer = pl.pallas_call(
    _tc_layer_body,
    out_shape=jax.ShapeDtypeStruct((N, D), jnp.float32),
)


def _tc_final_body(h_ref, batch_ref, l1w_ref, l1b_ref, l2w_ref, l2b_ref,
                   out_ref):
    gids = lax.broadcasted_iota(jnp.int32, (G, N), 0)
    onehot = (gids == batch_ref[...]).astype(jnp.float32)
    counts = jnp.sum(onehot, axis=1, keepdims=True)
    pooled = jnp.dot(onehot, h_ref[...],
                     preferred_element_type=jnp.float32)
    pooled = pooled / jnp.maximum(counts, 1.0)
    t = jnp.maximum(
        jnp.dot(pooled, l1w_ref[...], preferred_element_type=jnp.float32)
        + l1b_ref[...], 0.0)
    o = (jnp.dot(t, l2w_ref[...], preferred_element_type=jnp.float32)
         + l2b_ref[...])
    m = jnp.max(o, axis=1, keepdims=True)
    e = o - m
    out_ref[...] = e - jnp.log(jnp.sum(jnp.exp(e), axis=1, keepdims=True))


_tc_final = pl.pallas_call(
    _tc_final_body,
    out_shape=jax.ShapeDtypeStruct((G, 40), jnp.float32),
)


def kernel(x, edge_index, batch, params):
    pad = E_PAD - E
    src = jnp.concatenate(
        [edge_index[0], jnp.zeros((pad,), jnp.int32)]
    ).reshape(NW, NCHUNK, CHUNK)
    dst = jnp.concatenate(
        [edge_index[1], jnp.full((pad,), N, jnp.int32)]
    ).reshape(NW, NCHUNK, CHUNK)
    zeros = jnp.zeros((N, D), jnp.float32)
    sc_agg = _make_sc_agg()
    h = x
    for lp in params['convs']:
        agg = sc_agg(h, src, dst, zeros)
        h = _tc_layer(h, agg,
                      lp['W1'], lp['b1'].reshape(1, D),
                      lp['W2'], lp['b2'].reshape(1, D),
                      lp['gamma'].reshape(1, D), lp['beta'].reshape(1, D),
                      lp['eps'].reshape(1, 1))
    return _tc_final(h, batch.reshape(1, N),
                     params['lin1_W'], params['lin1_b'].reshape(1, D),
                     params['lin2_W'], params['lin2_b'].reshape(1, 40))


# speedup vs baseline: 2.7256x; 2.7256x over previous
import functools

import jax
import jax.numpy as jnp
from jax import lax
from jax.experimental import pallas as pl
from jax.experimental.pallas import tpu as pltpu
from jax.experimental.pallas import tpu_sc as plsc

N = 10000
E = 320000
D = 128
G = 64
NC = 2
NS = 16
NW = NC * NS
CHUNK = 100
NCHUNK = 100
EPW = CHUNK * NCHUNK
E_PAD = EPW * NW
ACC_ROWS = N
STRIPE = 640
TAIL = N - 15 * STRIPE

@functools.cache
def _make_sc_agg():
    mesh = plsc.VectorSubcoreMesh(core_axis_name="c", subcore_axis_name="s")

    @functools.partial(
        pl.kernel,
        mesh=mesh,
        out_type=jax.ShapeDtypeStruct((NC, N, D), jnp.float32),
        scratch_types=[
            pltpu.VMEM((NCHUNK, CHUNK), jnp.int32),
            pltpu.VMEM((NCHUNK, CHUNK), jnp.int32),
            pltpu.VMEM((CHUNK, D), jnp.float32),
            pltpu.VMEM_SHARED((ACC_ROWS, D), jnp.float32),
            pltpu.SemaphoreType.DMA,
        ],
    )
    def _sc_agg(h_hbm, src_hbm, dst_hbm, zeros_hbm, out_hbm,
                src_v, dst_v, rows_v, acc_sh, semg):
        c = lax.axis_index("c")
        s = lax.axis_index("s")
        w = c * NS + s

        pltpu.async_copy(src_hbm.at[w], src_v, semg)
        pltpu.async_copy(dst_hbm.at[w], dst_v, semg)

        @pl.when(s < 15)
        def _():
            pltpu.sync_copy(zeros_hbm.at[pl.ds(s * STRIPE, STRIPE)],
                            acc_sh.at[pl.ds(s * STRIPE, STRIPE)])

        @pl.when(s == 15)
        def _():
            pltpu.sync_copy(zeros_hbm.at[pl.ds(15 * STRIPE, TAIL)],
                            acc_sh.at[pl.ds(15 * STRIPE, TAIL)])

        pltpu.make_async_copy(src_hbm.at[w], src_v, semg).wait()
        pltpu.make_async_copy(dst_hbm.at[w], dst_v, semg).wait()
        plsc.subcore_barrier()

        def body(i, carry):
            pltpu.async_copy(h_hbm.at[src_v.at[i]], rows_v, semg).wait()
            pltpu.sync_copy(rows_v, acc_sh.at[dst_v.at[i]], add=True)
            return carry

        lax.fori_loop(0, NCHUNK, body, 0)
        plsc.subcore_barrier()

        @pl.when(s < 15)
        def _():
            pltpu.sync_copy(acc_sh.at[pl.ds(s * STRIPE, STRIPE)],
                            out_hbm.at[c, pl.ds(s * STRIPE, STRIPE)])

        @pl.when(s == 15)
        def _():
            pltpu.sync_copy(acc_sh.at[pl.ds(15 * STRIPE, TAIL)],
                            out_hbm.at[c, pl.ds(15 * STRIPE, TAIL)])

    return _sc_agg


def _tc_layer_body(h_ref, agg_ref, w1_ref, b1_ref, w2_ref, b2_ref,
                   gamma_ref, beta_ref, eps_ref, out_ref):
    z = (1.0 + eps_ref[0, 0]) * h_ref[...] + agg_ref[0] + agg_ref[1]
    z = jnp.maximum(
        jnp.dot(z, w1_ref[...], preferred_element_type=jnp.float32)
        + b1_ref[...], 0.0)
    z = jnp.maximum(
        jnp.dot(z, w2_ref[...], preferred_element_type=jnp.float32)
        + b2_ref[...], 0.0)
    mean = jnp.mean(z, axis=0, keepdims=True)
    var = jnp.mean((z - mean) * (z - mean), axis=0, keepdims=True)
    out_ref[...] = ((z - mean) * lax.rsqrt(var + 1e-5) * gamma_ref[...]
                    + beta_ref[...])


_tc_layer = pl.pallas_call(
    _tc_layer_body,
    out_shape=jax.ShapeDtypeStruct((N, D), jnp.float32),
)


def _tc_final_body(h_ref, batch_ref, l1w_ref, l1b_ref, l2w_ref, l2b_ref,
                   out_ref):
    gids = lax.broadcasted_iota(jnp.int32, (G, N), 0)
    onehot = (gids == batch_ref[...]).astype(jnp.float32)
    counts = jnp.sum(onehot, axis=1, keepdims=True)
    pooled = jnp.dot(onehot, h_ref[...],
                     preferred_element_type=jnp.float32)
    pooled = pooled / jnp.maximum(counts, 1.0)
    t = jnp.maximum(
        jnp.dot(pooled, l1w_ref[...], preferred_element_type=jnp.float32)
        + l1b_ref[...], 0.0)
    o = (jnp.dot(t, l2w_ref[...], preferred_element_type=jnp.float32)
         + l2b_ref[...])
    m = jnp.max(o, axis=1, keepdims=True)
    e = o - m
    out_ref[...] = e - jnp.log(jnp.sum(jnp.exp(e), axis=1, keepdims=True))


_tc_final = pl.pallas_call(
    _tc_final_body,
    out_shape=jax.ShapeDtypeStruct((G, 40), jnp.float32),
)


def kernel(x, edge_index, batch, params):
    pad = E_PAD - E
    src = jnp.concatenate(
        [edge_index[0], jnp.zeros((pad,), jnp.int32)]
    ).reshape(NW, NCHUNK, CHUNK)
    dst = jnp.concatenate(
        [edge_index[1], jnp.full((pad,), N, jnp.int32)]
    ).reshape(NW, NCHUNK, CHUNK)
    zeros = jnp.zeros((N, D), jnp.float32)
    sc_agg = _make_sc_agg()
    h = x
    for lp in params['convs']:
        agg = sc_agg(h, src, dst, zeros)
        h = _tc_layer(h, agg,
                      lp['W1'], lp['b1'].reshape(1, D),
                      lp['W2'], lp['b2'].reshape(1, D),
                      lp['gamma'].reshape(1, D), lp['beta'].reshape(1, D),
                      lp['eps'].reshape(1, 1))
    return _tc_final(h, batch.reshape(1, N),
                     params['lin1_W'], params['lin1_b'].reshape(1, D),
                     params['lin2_W'], params['lin2_b'].reshape(1, 40))


# speedup vs baseline: 2.9086x; 1.0672x over previous
import functools

import jax
import jax.numpy as jnp
from jax import lax
from jax.experimental import pallas as pl
from jax.experimental.pallas import tpu as pltpu
from jax.experimental.pallas import tpu_sc as plsc

N = 10000
E = 320000
D = 128
G = 64
NC = 2
NS = 16
NW = NC * NS
CHUNK = 125
NCHUNK = 80
EPW = CHUNK * NCHUNK
E_PAD = EPW * NW
ACC_ROWS = N
STRIPE = 640
TAIL = N - 15 * STRIPE

@functools.cache
def _make_sc_agg():
    mesh = plsc.VectorSubcoreMesh(core_axis_name="c", subcore_axis_name="s")

    @functools.partial(
        pl.kernel,
        mesh=mesh,
        out_type=jax.ShapeDtypeStruct((NC, N, D), jnp.float32),
        scratch_types=[
            pltpu.VMEM((NCHUNK, CHUNK), jnp.int32),
            pltpu.VMEM((NCHUNK, CHUNK), jnp.int32),
            pltpu.VMEM((CHUNK, D), jnp.float32),
            pltpu.VMEM_SHARED((ACC_ROWS, D), jnp.float32),
            pltpu.SemaphoreType.DMA,
        ],
    )
    def _sc_agg(h_hbm, src_hbm, dst_hbm, zeros_hbm, out_hbm,
                src_v, dst_v, rows_v, acc_sh, semg):
        c = lax.axis_index("c")
        s = lax.axis_index("s")
        w = c * NS + s

        pltpu.async_copy(src_hbm.at[w], src_v, semg)
        pltpu.async_copy(dst_hbm.at[w], dst_v, semg)

        @pl.when(s < 15)
        def _():
            pltpu.sync_copy(zeros_hbm.at[pl.ds(s * STRIPE, STRIPE)],
                            acc_sh.at[pl.ds(s * STRIPE, STRIPE)])

        @pl.when(s == 15)
        def _():
            pltpu.sync_copy(zeros_hbm.at[pl.ds(15 * STRIPE, TAIL)],
                            acc_sh.at[pl.ds(15 * STRIPE, TAIL)])

        pltpu.make_async_copy(src_hbm.at[w], src_v, semg).wait()
        pltpu.make_async_copy(dst_hbm.at[w], dst_v, semg).wait()
        plsc.subcore_barrier()

        def body(i, carry):
            pltpu.async_copy(h_hbm.at[src_v.at[i]], rows_v, semg).wait()
            pltpu.sync_copy(rows_v, acc_sh.at[dst_v.at[i]], add=True)
            return carry

        lax.fori_loop(0, NCHUNK, body, 0)
        plsc.subcore_barrier()

        @pl.when(s < 15)
        def _():
            pltpu.sync_copy(acc_sh.at[pl.ds(s * STRIPE, STRIPE)],
                            out_hbm.at[c, pl.ds(s * STRIPE, STRIPE)])

        @pl.when(s == 15)
        def _():
            pltpu.sync_copy(acc_sh.at[pl.ds(15 * STRIPE, TAIL)],
                            out_hbm.at[c, pl.ds(15 * STRIPE, TAIL)])

    return _sc_agg


def _tc_layer_body(h_ref, agg_ref, w1_ref, b1_ref, w2_ref, b2_ref,
                   gamma_ref, beta_ref, eps_ref, out_ref):
    z = (1.0 + eps_ref[0, 0]) * h_ref[...] + agg_ref[0] + agg_ref[1]
    z = jnp.maximum(
        jnp.dot(z, w1_ref[...], preferred_element_type=jnp.float32)
        + b1_ref[...], 0.0)
    z = jnp.maximum(
        jnp.dot(z, w2_ref[...], preferred_element_type=jnp.float32)
        + b2_ref[...], 0.0)
    mean = jnp.mean(z, axis=0, keepdims=True)
    var = jnp.mean((z - mean) * (z - mean), axis=0, keepdims=True)
    out_ref[...] = ((z - mean) * lax.rsqrt(var + 1e-5) * gamma_ref[...]
                    + beta_ref[...])


_tc_layer = pl.pallas_call(
    _tc_layer_body,
    out_shape=jax.ShapeDtypeStruct((N, D), jnp.float32),
)


def _tc_final_body(h_ref, batch_ref, l1w_ref, l1b_ref, l2w_ref, l2b_ref,
                   out_ref):
    gids = lax.broadcasted_iota(jnp.int32, (G, N), 0)
    onehot = (gids == batch_ref[...]).astype(jnp.float32)
    counts = jnp.sum(onehot, axis=1, keepdims=True)
    pooled = jnp.dot(onehot, h_ref[...],
                     preferred_element_type=jnp.float32)
    pooled = pooled / jnp.maximum(counts, 1.0)
    t = jnp.maximum(
        jnp.dot(pooled, l1w_ref[...], preferred_element_type=jnp.float32)
        + l1b_ref[...], 0.0)
    o = (jnp.dot(t, l2w_ref[...], preferred_element_type=jnp.float32)
         + l2b_ref[...])
    m = jnp.max(o, axis=1, keepdims=True)
    e = o - m
    out_ref[...] = e - jnp.log(jnp.sum(jnp.exp(e), axis=1, keepdims=True))


_tc_final = pl.pallas_call(
    _tc_final_body,
    out_shape=jax.ShapeDtypeStruct((G, 40), jnp.float32),
)


def kernel(x, edge_index, batch, params):
    pad = E_PAD - E
    src = jnp.concatenate(
        [edge_index[0], jnp.zeros((pad,), jnp.int32)]
    ).reshape(NW, NCHUNK, CHUNK)
    dst = jnp.concatenate(
        [edge_index[1], jnp.full((pad,), N, jnp.int32)]
    ).reshape(NW, NCHUNK, CHUNK)
    zeros = jnp.zeros((N, D), jnp.float32)
    sc_agg = _make_sc_agg()
    h = x
    for lp in params['convs']:
        agg = sc_agg(h, src, dst, zeros)
        h = _tc_layer(h, agg,
                      lp['W1'], lp['b1'].reshape(1, D),
                      lp['W2'], lp['b2'].reshape(1, D),
                      lp['gamma'].reshape(1, D), lp['beta'].reshape(1, D),
                      lp['eps'].reshape(1, 1))
    return _tc_final(h, batch.reshape(1, N),
                     params['lin1_W'], params['lin1_b'].reshape(1, D),
                     params['lin2_W'], params['lin2_b'].reshape(1, 40))


# speedup vs baseline: 3.7178x; 1.2782x over previous
import functools

import jax
import jax.numpy as jnp
from jax import lax
from jax.experimental import pallas as pl
from jax.experimental.pallas import tpu as pltpu
from jax.experimental.pallas import tpu_sc as plsc

N = 10000
E = 320000
D = 128
G = 64
NC = 2
NS = 16
NW = NC * NS
CHUNK = 125
NCHUNK = 80
HALFC = NCHUNK // 2
EPW = CHUNK * NCHUNK
E_PAD = EPW * NW
ACC_ROWS = N
SCAT_BYTES = CHUNK * D * 4
STRIPE = 640
TAIL = N - 15 * STRIPE

@functools.cache
def _make_sc_agg():
    mesh = plsc.VectorSubcoreMesh(core_axis_name="c", subcore_axis_name="s")

    @functools.partial(
        pl.kernel,
        mesh=mesh,
        out_type=jax.ShapeDtypeStruct((NC, N, D), jnp.float32),
        scratch_types=[
            pltpu.VMEM((HALFC, CHUNK), jnp.int32),
            pltpu.VMEM((HALFC, CHUNK), jnp.int32),
            pltpu.VMEM((CHUNK, D), jnp.float32),
            pltpu.VMEM((CHUNK, D), jnp.float32),
            pltpu.VMEM_SHARED((ACC_ROWS, D), jnp.float32),
            pltpu.SemaphoreType.DMA,
            pltpu.SemaphoreType.DMA,
            pltpu.SemaphoreType.DMA,
        ],
    )
    def _sc_agg(h_hbm, src_hbm, dst_hbm, zeros_hbm, out_hbm,
                src_v, dst_v, rows0_v, rows1_v, acc_sh,
                semg, sems0, sems1):
        c = lax.axis_index("c")
        s = lax.axis_index("s")
        w = c * NS + s

        pltpu.async_copy(src_hbm.at[w, 0], src_v, semg)
        pltpu.async_copy(dst_hbm.at[w, 0], dst_v, semg)

        @pl.when(s < 15)
        def _():
            pltpu.sync_copy(zeros_hbm.at[pl.ds(s * STRIPE, STRIPE)],
                            acc_sh.at[pl.ds(s * STRIPE, STRIPE)])

        @pl.when(s == 15)
        def _():
            pltpu.sync_copy(zeros_hbm.at[pl.ds(15 * STRIPE, TAIL)],
                            acc_sh.at[pl.ds(15 * STRIPE, TAIL)])

        pltpu.make_async_copy(src_hbm.at[w, 0], src_v, semg).wait()
        pltpu.make_async_copy(dst_hbm.at[w, 0], dst_v, semg).wait()
        plsc.subcore_barrier()

        for p in range(2):
            pltpu.async_copy(h_hbm.at[src_v.at[0]], rows0_v, semg).wait()
            pltpu.async_copy(rows0_v, acc_sh.at[dst_v.at[0]],
                             sems0, add=True)
            pltpu.async_copy(h_hbm.at[src_v.at[1]], rows1_v, semg).wait()
            pltpu.async_copy(rows1_v, acc_sh.at[dst_v.at[1]],
                             sems1, add=True)

            def body(j, carry):
                i0 = 2 * j
                i1 = i0 + 1
                pltpu.make_async_copy(rows0_v, acc_sh.at[dst_v.at[i0]],
                                      sems0).wait()
                pltpu.async_copy(h_hbm.at[src_v.at[i0]], rows0_v,
                                 semg).wait()
                pltpu.async_copy(rows0_v, acc_sh.at[dst_v.at[i0]],
                                 sems0, add=True)
                pltpu.make_async_copy(rows1_v, acc_sh.at[dst_v.at[i1]],
                                      sems1).wait()
                pltpu.async_copy(h_hbm.at[src_v.at[i1]], rows1_v,
                                 semg).wait()
                pltpu.async_copy(rows1_v, acc_sh.at[dst_v.at[i1]],
                                 sems1, add=True)
                return carry

            lax.fori_loop(1, HALFC // 2, body, 0)
            pltpu.make_async_copy(rows0_v, acc_sh.at[dst_v.at[0]],
                                  sems0).wait()
            pltpu.make_async_copy(rows1_v, acc_sh.at[dst_v.at[1]],
                                  sems1).wait()
            if p == 0:
                pltpu.sync_copy(src_hbm.at[w, 1], src_v)
                pltpu.sync_copy(dst_hbm.at[w, 1], dst_v)

        plsc.subcore_barrier()

        @pl.when(s < 15)
        def _():
            pltpu.sync_copy(acc_sh.at[pl.ds(s * STRIPE, STRIPE)],
                            out_hbm.at[c, pl.ds(s * STRIPE, STRIPE)])

        @pl.when(s == 15)
        def _():
            pltpu.sync_copy(acc_sh.at[pl.ds(15 * STRIPE, TAIL)],
                            out_hbm.at[c, pl.ds(15 * STRIPE, TAIL)])

    return _sc_agg


def _tc_layer_body(h_ref, agg_ref, w1_ref, b1_ref, w2_ref, b2_ref,
                   gamma_ref, beta_ref, eps_ref, out_ref):
    z = (1.0 + eps_ref[0, 0]) * h_ref[...] + agg_ref[0] + agg_ref[1]
    z = jnp.maximum(
        jnp.dot(z, w1_ref[...], preferred_element_type=jnp.float32)
        + b1_ref[...], 0.0)
    z = jnp.maximum(
        jnp.dot(z, w2_ref[...], preferred_element_type=jnp.float32)
        + b2_ref[...], 0.0)
    mean = jnp.mean(z, axis=0, keepdims=True)
    var = jnp.mean((z - mean) * (z - mean), axis=0, keepdims=True)
    out_ref[...] = ((z - mean) * lax.rsqrt(var + 1e-5) * gamma_ref[...]
                    + beta_ref[...])


_tc_layer = pl.pallas_call(
    _tc_layer_body,
    out_shape=jax.ShapeDtypeStruct((N, D), jnp.float32),
)


def _tc_final_body(h_ref, batch_ref, l1w_ref, l1b_ref, l2w_ref, l2b_ref,
                   out_ref):
    gids = lax.broadcasted_iota(jnp.int32, (G, N), 0)
    onehot = (gids == batch_ref[...]).astype(jnp.float32)
    counts = jnp.sum(onehot, axis=1, keepdims=True)
    pooled = jnp.dot(onehot, h_ref[...],
                     preferred_element_type=jnp.float32)
    pooled = pooled / jnp.maximum(counts, 1.0)
    t = jnp.maximum(
        jnp.dot(pooled, l1w_ref[...], preferred_element_type=jnp.float32)
        + l1b_ref[...], 0.0)
    o = (jnp.dot(t, l2w_ref[...], preferred_element_type=jnp.float32)
         + l2b_ref[...])
    m = jnp.max(o, axis=1, keepdims=True)
    e = o - m
    out_ref[...] = e - jnp.log(jnp.sum(jnp.exp(e), axis=1, keepdims=True))


_tc_final = pl.pallas_call(
    _tc_final_body,
    out_shape=jax.ShapeDtypeStruct((G, 40), jnp.float32),
)


def kernel(x, edge_index, batch, params):
    src = edge_index[0].reshape(NW, 2, HALFC, CHUNK)
    dst = edge_index[1].reshape(NW, 2, HALFC, CHUNK)
    zeros = jnp.zeros((N, D), jnp.float32)
    sc_agg = _make_sc_agg()
    h = x
    for lp in params['convs']:
        agg = sc_agg(h, src, dst, zeros)
        h = _tc_layer(h, agg,
                      lp['W1'], lp['b1'].reshape(1, D),
                      lp['W2'], lp['b2'].reshape(1, D),
                      lp['gamma'].reshape(1, D), lp['beta'].reshape(1, D),
                      lp['eps'].reshape(1, 1))
    return _tc_final(h, batch.reshape(1, N),
                     params['lin1_W'], params['lin1_b'].reshape(1, D),
                     params['lin2_W'], params['lin2_b'].reshape(1, 40))


# speedup vs baseline: 3.7516x; 1.0091x over previous
import functools

import jax
import jax.numpy as jnp
from jax import lax
from jax.experimental import pallas as pl
from jax.experimental.pallas import tpu as pltpu
from jax.experimental.pallas import tpu_sc as plsc

N = 10000
E = 320000
D = 128
G = 64
NC = 2
NS = 16
NW = NC * NS
CHUNK = 125
NCHUNK = 80
HALFC = NCHUNK // 2
EPW = CHUNK * NCHUNK
E_PAD = EPW * NW
ACC_ROWS = N
SCAT_BYTES = CHUNK * D * 4
STRIPE = 640
TAIL = N - 15 * STRIPE

@functools.cache
def _make_sc_agg():
    mesh = plsc.VectorSubcoreMesh(core_axis_name="c", subcore_axis_name="s")

    @functools.partial(
        pl.kernel,
        mesh=mesh,
        out_type=jax.ShapeDtypeStruct((NC, N, D), jnp.float32),
        scratch_types=[
            pltpu.VMEM((HALFC, CHUNK), jnp.int32),
            pltpu.VMEM((HALFC, CHUNK), jnp.int32),
            pltpu.VMEM((CHUNK, D), jnp.float32),
            pltpu.VMEM((CHUNK, D), jnp.float32),
            pltpu.VMEM_SHARED((ACC_ROWS, D), jnp.float32),
            pltpu.SemaphoreType.DMA,
            pltpu.SemaphoreType.DMA,
            pltpu.SemaphoreType.DMA,
        ],
    )
    def _sc_agg(h_hbm, src_hbm, dst_hbm, zeros_hbm, out_hbm,
                src_v, dst_v, rows0_v, rows1_v, acc_sh,
                semg, sems0, sems1):
        c = lax.axis_index("c")
        s = lax.axis_index("s")
        w = c * NS + s

        pltpu.async_copy(src_hbm.at[w, 0], src_v, semg)
        pltpu.async_copy(dst_hbm.at[w, 0], dst_v, semg)

        @pl.when(s < 15)
        def _():
            pltpu.sync_copy(zeros_hbm.at[pl.ds(s * STRIPE, STRIPE)],
                            acc_sh.at[pl.ds(s * STRIPE, STRIPE)])

        @pl.when(s == 15)
        def _():
            pltpu.sync_copy(zeros_hbm.at[pl.ds(15 * STRIPE, TAIL)],
                            acc_sh.at[pl.ds(15 * STRIPE, TAIL)])

        pltpu.make_async_copy(src_hbm.at[w, 0], src_v, semg).wait()
        pltpu.make_async_copy(dst_hbm.at[w, 0], dst_v, semg).wait()
        plsc.subcore_barrier()

        for p in range(2):
            pltpu.async_copy(h_hbm.at[src_v.at[0]], rows0_v, semg).wait()
            pltpu.async_copy(rows0_v, acc_sh.at[dst_v.at[0]],
                             sems0, add=True)
            pltpu.async_copy(h_hbm.at[src_v.at[1]], rows1_v, semg).wait()
            pltpu.async_copy(rows1_v, acc_sh.at[dst_v.at[1]],
                             sems1, add=True)

            def body(j, carry):
                i0 = 2 * j
                i1 = i0 + 1
                pltpu.make_async_copy(rows0_v, acc_sh.at[dst_v.at[i0]],
                                      sems0).wait()
                pltpu.async_copy(h_hbm.at[src_v.at[i0]], rows0_v,
                                 semg).wait()
                pltpu.async_copy(rows0_v, acc_sh.at[dst_v.at[i0]],
                                 sems0, add=True)
                pltpu.make_async_copy(rows1_v, acc_sh.at[dst_v.at[i1]],
                                      sems1).wait()
                pltpu.async_copy(h_hbm.at[src_v.at[i1]], rows1_v,
                                 semg).wait()
                pltpu.async_copy(rows1_v, acc_sh.at[dst_v.at[i1]],
                                 sems1, add=True)
                return carry

            lax.fori_loop(1, HALFC // 2, body, 0)
            pltpu.make_async_copy(rows0_v, acc_sh.at[dst_v.at[0]],
                                  sems0).wait()
            pltpu.make_async_copy(rows1_v, acc_sh.at[dst_v.at[1]],
                                  sems1).wait()
            if p == 0:
                pltpu.sync_copy(src_hbm.at[w, 1], src_v)
                pltpu.sync_copy(dst_hbm.at[w, 1], dst_v)

        plsc.subcore_barrier()

        @pl.when(s < 15)
        def _():
            pltpu.sync_copy(acc_sh.at[pl.ds(s * STRIPE, STRIPE)],
                            out_hbm.at[c, pl.ds(s * STRIPE, STRIPE)])

        @pl.when(s == 15)
        def _():
            pltpu.sync_copy(acc_sh.at[pl.ds(15 * STRIPE, TAIL)],
                            out_hbm.at[c, pl.ds(15 * STRIPE, TAIL)])

    return _sc_agg


def _tc_layer_body(h_ref, agg_ref, w1_ref, b1_ref, w2_ref, b2_ref,
                   gamma_ref, beta_ref, eps_ref, out_ref):
    z = (1.0 + eps_ref[0, 0]) * h_ref[...] + agg_ref[0] + agg_ref[1]
    z = jnp.maximum(
        jnp.dot(z, w1_ref[...], preferred_element_type=jnp.float32)
        + b1_ref[...], 0.0)
    z = jnp.maximum(
        jnp.dot(z, w2_ref[...], preferred_element_type=jnp.float32)
        + b2_ref[...], 0.0)
    mean = jnp.mean(z, axis=0, keepdims=True)
    var = jnp.mean((z - mean) * (z - mean), axis=0, keepdims=True)
    out_ref[...] = ((z - mean) * lax.rsqrt(var + 1e-5) * gamma_ref[...]
                    + beta_ref[...])


_tc_layer = pl.pallas_call(
    _tc_layer_body,
    out_shape=jax.ShapeDtypeStruct((N, D), jnp.float32),
)


def _tc_last_body(h_ref, agg_ref, w1_ref, b1_ref, w2_ref, b2_ref,
                  gamma_ref, beta_ref, eps_ref, batch_ref,
                  l1w_ref, l1b_ref, l2w_ref, l2b_ref, out_ref):
    z = (1.0 + eps_ref[0, 0]) * h_ref[...] + agg_ref[0] + agg_ref[1]
    z = jnp.maximum(
        jnp.dot(z, w1_ref[...], preferred_element_type=jnp.float32)
        + b1_ref[...], 0.0)
    z = jnp.maximum(
        jnp.dot(z, w2_ref[...], preferred_element_type=jnp.float32)
        + b2_ref[...], 0.0)
    mean = jnp.mean(z, axis=0, keepdims=True)
    var = jnp.mean((z - mean) * (z - mean), axis=0, keepdims=True)
    h3 = ((z - mean) * lax.rsqrt(var + 1e-5) * gamma_ref[...]
          + beta_ref[...])
    gids = lax.broadcasted_iota(jnp.int32, (G, N), 0)
    onehot = (gids == batch_ref[...]).astype(jnp.float32)
    counts = jnp.sum(onehot, axis=1, keepdims=True)
    pooled = jnp.dot(onehot, h3, preferred_element_type=jnp.float32)
    pooled = pooled / jnp.maximum(counts, 1.0)
    t = jnp.maximum(
        jnp.dot(pooled, l1w_ref[...], preferred_element_type=jnp.float32)
        + l1b_ref[...], 0.0)
    o = (jnp.dot(t, l2w_ref[...], preferred_element_type=jnp.float32)
         + l2b_ref[...])
    m = jnp.max(o, axis=1, keepdims=True)
    e = o - m
    out_ref[...] = e - jnp.log(jnp.sum(jnp.exp(e), axis=1, keepdims=True))


_tc_last = pl.pallas_call(
    _tc_last_body,
    out_shape=jax.ShapeDtypeStruct((G, 40), jnp.float32),
)


def kernel(x, edge_index, batch, params):
    src = edge_index[0].reshape(NW, 2, HALFC, CHUNK)
    dst = edge_index[1].reshape(NW, 2, HALFC, CHUNK)
    zeros = jnp.zeros((N, D), jnp.float32)
    sc_agg = _make_sc_agg()
    h = x
    for lp in params['convs'][:-1]:
        agg = sc_agg(h, src, dst, zeros)
        h = _tc_layer(h, agg,
                      lp['W1'], lp['b1'].reshape(1, D),
                      lp['W2'], lp['b2'].reshape(1, D),
                      lp['gamma'].reshape(1, D), lp['beta'].reshape(1, D),
                      lp['eps'].reshape(1, 1))
    lp = params['convs'][-1]
    agg = sc_agg(h, src, dst, zeros)
    return _tc_last(h, agg,
                    lp['W1'], lp['b1'].reshape(1, D),
                    lp['W2'], lp['b2'].reshape(1, D),
                    lp['gamma'].reshape(1, D), lp['beta'].reshape(1, D),
                    lp['eps'].reshape(1, 1), batch.reshape(1, N),
                    params['lin1_W'], params['lin1_b'].reshape(1, D),
                    params['lin2_W'], params['lin2_b'].reshape(1, 40))
